# Initial kernel scaffold; baseline (speedup 1.0000x reference)
import jax
import jax.numpy as jnp
from jax.experimental import pallas as pl


def kernel(x, edge_index, label, prior_labels, W1, b1, W2, b2, W3, b3, W4, b4, W5, b5):
    raise NotImplementedError("write your pallas kernel here")


# speedup vs baseline: 22.9376x; 22.9376x over previous
import functools

import jax
import jax.numpy as jnp
from jax import lax
from jax.experimental import pallas as pl
from jax.experimental.pallas import tpu as pltpu
from jax.experimental.pallas import tpu_sc as plsc

N_NODES = 10000
N_EDGES = 320000
D_FEAT = 128
D_HID = 64

NC = 2
NS = 16
NW = NC * NS
EW = N_EDGES // NW
C = 100
NCH = EW // C
N_PAD = 10240
ROWS = N_PAD // NS
DROWS = N_PAD // NS

_SC_PARAMS = pltpu.CompilerParams(use_tc_tiling_on_sc=False)


def _mesh():
    return plsc.VectorSubcoreMesh(core_axis_name="c", subcore_axis_name="s")


def _make_degree():
    @functools.partial(
        pl.kernel,
        out_type=jax.ShapeDtypeStruct((NC, N_PAD), jnp.float32),
        mesh=_mesh(),
        compiler_params=_SC_PARAMS,
        scratch_types=[
            pltpu.VMEM((NCH, C), jnp.int32),
            pltpu.VMEM((112,), jnp.float32),
            pltpu.VMEM_SHARED((N_PAD,), jnp.float32),
        ],
    )
    def degree(dst2_hbm, z_hbm, out_hbm, didx, ones_v, acc):
        c = lax.axis_index("c")
        s = lax.axis_index("s")
        w = s * NC + c
        pltpu.sync_copy(z_hbm.at[pl.ds(s * DROWS, DROWS)],
                        acc.at[pl.ds(s * DROWS, DROWS)])
        pltpu.sync_copy(dst2_hbm.at[w], didx)
        for i in range(7):
            ones_v[pl.ds(i * 16, 16)] = jnp.ones((16,), jnp.float32)
        plsc.subcore_barrier()

        def step(k, carry):
            pltpu.sync_copy(ones_v.at[pl.ds(0, C)], acc.at[didx.at[k]], add=True)
            return carry

        lax.fori_loop(0, NCH, step, 0)
        plsc.subcore_barrier()
        pltpu.sync_copy(acc.at[pl.ds(s * DROWS, DROWS)],
                        out_hbm.at[c, pl.ds(s * DROWS, DROWS)])

    return degree


def _make_propagate(D):
    @functools.partial(
        pl.kernel,
        out_type=jax.ShapeDtypeStruct((NC, N_PAD, D), jnp.float32),
        mesh=_mesh(),
        compiler_params=_SC_PARAMS,
        scratch_types=[
            pltpu.VMEM((NCH, C), jnp.int32),
            pltpu.VMEM((NCH, C), jnp.int32),
            pltpu.VMEM((C, D), jnp.float32),
            pltpu.VMEM((C, D), jnp.float32),
            pltpu.VMEM_SHARED((N_PAD, D), jnp.float32),
            pltpu.SemaphoreType.DMA,
            pltpu.SemaphoreType.DMA,
        ],
    )
    def propagate(p_hbm, src2_hbm, dst2_hbm, z_hbm, out_hbm,
                  sidx, didx, r0, r1, acc, sem0, sem1):
        c = lax.axis_index("c")
        s = lax.axis_index("s")
        w = s * NC + c
        pltpu.sync_copy(z_hbm.at[pl.ds(s * ROWS, ROWS)],
                        acc.at[pl.ds(s * ROWS, ROWS)])
        pltpu.sync_copy(src2_hbm.at[w], sidx)
        pltpu.sync_copy(dst2_hbm.at[w], didx)
        plsc.subcore_barrier()

        pltpu.async_copy(p_hbm.at[sidx.at[0]], r0, sem0)
        pltpu.async_copy(p_hbm.at[sidx.at[1]], r1, sem1)

        def step(i, carry):
            kk = 2 * i
            pltpu.make_async_copy(p_hbm.at[sidx.at[kk]], r0, sem0).wait()
            pltpu.sync_copy(r0, acc.at[didx.at[kk]], add=True)
            pltpu.async_copy(p_hbm.at[sidx.at[kk + 2]], r0, sem0)
            pltpu.make_async_copy(p_hbm.at[sidx.at[kk + 1]], r1, sem1).wait()
            pltpu.sync_copy(r1, acc.at[didx.at[kk + 1]], add=True)
            pltpu.async_copy(p_hbm.at[sidx.at[kk + 3]], r1, sem1)
            return carry

        lax.fori_loop(0, (NCH - 2) // 2, step, 0)
        pltpu.make_async_copy(p_hbm.at[sidx.at[NCH - 2]], r0, sem0).wait()
        pltpu.sync_copy(r0, acc.at[didx.at[NCH - 2]], add=True)
        pltpu.make_async_copy(p_hbm.at[sidx.at[NCH - 1]], r1, sem1).wait()
        pltpu.sync_copy(r1, acc.at[didx.at[NCH - 1]], add=True)

        plsc.subcore_barrier()
        pltpu.sync_copy(acc.at[pl.ds(s * ROWS, ROWS)],
                        out_hbm.at[c, pl.ds(s * ROWS, ROWS)])

    return propagate


BM = 1000


def _dinv_body(deg_ref, out_ref):
    d = deg_ref[0, :] + deg_ref[1, :] + 1.0
    out_ref[...] = jax.lax.rsqrt(d)[:, None]


def _dinv_kernel(deg):
    return pl.pallas_call(
        _dinv_body,
        out_shape=jax.ShapeDtypeStruct((N_PAD, 1), jnp.float32),
    )(deg)


def _scale_mm_body(dinv_ref, x_ref, w_ref, out_ref):
    t = jnp.dot(x_ref[...], w_ref[...], preferred_element_type=jnp.float32)
    out_ref[...] = dinv_ref[...] * t


def _scale_mm(dinv, x, w):
    m, k = x.shape
    n = w.shape[1]
    return pl.pallas_call(
        _scale_mm_body,
        grid=(m // BM,),
        in_specs=[
            pl.BlockSpec((BM, 1), lambda i: (i, 0)),
            pl.BlockSpec((BM, k), lambda i: (i, 0)),
            pl.BlockSpec((k, n), lambda i: (0, 0)),
        ],
        out_specs=pl.BlockSpec((BM, n), lambda i: (i, 0)),
        out_shape=jax.ShapeDtypeStruct((m, n), jnp.float32),
    )(dinv, x, w)


def _epilogue_body(nw, dinv_ref, a0_ref, a1_ref, p_ref, b_ref, *rest):
    w_refs = rest[:nw]
    out_refs = rest[nw:]
    dinv = dinv_ref[...]
    h = dinv * (a0_ref[0] + a1_ref[0] + p_ref[...]) + b_ref[...]
    h = jnp.maximum(h, 0.0)
    if nw == 0:
        out_refs[0][...] = h
    else:
        for w_ref, o_ref in zip(w_refs, out_refs):
            t = jnp.dot(h, w_ref[...], preferred_element_type=jnp.float32)
            out_refs_dinv = dinv * t
            o_ref[...] = out_refs_dinv


def _epilogue(dinv, a, p, b, ws):
    m, d = p.shape
    nw = len(ws)
    outs = ([jax.ShapeDtypeStruct((m, d), jnp.float32)] if nw == 0 else
            [jax.ShapeDtypeStruct((m, w.shape[1]), jnp.float32) for w in ws])
    w_specs = [pl.BlockSpec(w.shape, lambda i: (0, 0)) for w in ws]
    out_specs = ([pl.BlockSpec((BM, d), lambda i: (i, 0))] if nw == 0 else
                 [pl.BlockSpec((BM, w.shape[1]), lambda i: (i, 0)) for w in ws])
    res = pl.pallas_call(
        functools.partial(_epilogue_body, nw),
        grid=(m // BM,),
        in_specs=[
            pl.BlockSpec((BM, 1), lambda i: (i, 0)),
            pl.BlockSpec((1, BM, d), lambda i: (0, i, 0)),
            pl.BlockSpec((1, BM, d), lambda i: (1, i, 0)),
            pl.BlockSpec((BM, d), lambda i: (i, 0)),
            pl.BlockSpec((1, d), lambda i: (0, 0)),
            *w_specs,
        ],
        out_specs=out_specs,
        out_shape=outs,
    )(dinv, a, a, p, b, *ws)
    return res


BN = 1024


def _sst_body(sr_ref, sc_ref, out_ref):
    out_ref[...] = lax.dot_general(
        sr_ref[...], sc_ref[...],
        dimension_numbers=(((1,), (1,)), ((), ())),
        preferred_element_type=jnp.float32,
    )


def _s_st(s):
    m = s.shape[0]
    d = s.shape[1]
    return pl.pallas_call(
        _sst_body,
        grid=(m // BM, pl.cdiv(m, BN)),
        in_specs=[
            pl.BlockSpec((BM, d), lambda i, j: (i, 0)),
            pl.BlockSpec((BN, d), lambda i, j: (j, 0)),
        ],
        out_specs=pl.BlockSpec((BM, BN), lambda i, j: (i, j)),
        out_shape=jax.ShapeDtypeStruct((m, m), jnp.float32),
    )(s, s)


def kernel(x, edge_index, label, prior_labels, W1, b1, W2, b2, W3, b3, W4, b4, W5, b5):
    ei = edge_index.astype(jnp.int32)
    src2 = ei[0].reshape(NW, NCH, C)
    dst2 = ei[1].reshape(NW, NCH, C)

    z1 = jnp.zeros((N_PAD,), jnp.float32)
    z64 = jnp.zeros((N_PAD, D_HID), jnp.float32)
    z128 = jnp.zeros((N_PAD, D_FEAT), jnp.float32)

    deg = _make_degree()(dst2, z1)
    dinv = _dinv_kernel(deg)

    prop64 = _make_propagate(D_HID)
    prop128 = _make_propagate(D_FEAT)

    b1r = b1.reshape(1, -1)
    b2r = b2.reshape(1, -1)
    b3r = b3.reshape(1, -1)
    b4r = b4.reshape(1, -1)
    b5r = b5.reshape(1, -1)

    p1 = _scale_mm(dinv, x, W1)
    a1 = prop64(p1, src2, dst2, z64)
    (p2,) = _epilogue(dinv, a1, p1, b1r, [W2])

    a2 = prop64(p2, src2, dst2, z64)
    p3, p5 = _epilogue(dinv, a2, p2, b2r, [W3, W5])

    a3 = prop64(p3, src2, dst2, z64)
    (p4,) = _epilogue(dinv, a3, p3, b3r, [W4])
    a4 = prop128(p4, src2, dst2, z128)
    (x_hat,) = _epilogue(dinv, a4, p4, b4r, [])

    a5 = prop64(p5, src2, dst2, z64)
    (s,) = _epilogue(dinv, a5, p5, b5r, [])
    struct = _s_st(s)

    return (struct, x_hat, edge_index)


# speedup vs baseline: 25.8782x; 1.1282x over previous
import functools

import jax
import jax.numpy as jnp
from jax import lax
from jax.experimental import pallas as pl
from jax.experimental.pallas import tpu as pltpu
from jax.experimental.pallas import tpu_sc as plsc

N_NODES = 10000
N_EDGES = 320000
D_FEAT = 128
D_HID = 64

NC = 2
NS = 16
NW = NC * NS
EW = N_EDGES // NW
N_PAD = 10240
ROWS = N_PAD // NS
DROWS = N_PAD // NS

_SC_PARAMS = pltpu.CompilerParams(use_tc_tiling_on_sc=False)


def _mesh():
    return plsc.VectorSubcoreMesh(core_axis_name="c", subcore_axis_name="s")


C_DEG = 625
NCH_DEG = EW // C_DEG


def _make_degree():
    @functools.partial(
        pl.kernel,
        out_type=jax.ShapeDtypeStruct((NC, N_PAD), jnp.float32),
        mesh=_mesh(),
        compiler_params=_SC_PARAMS,
        scratch_types=[
            pltpu.VMEM((NCH_DEG, C_DEG), jnp.int32),
            pltpu.VMEM((C_DEG,), jnp.float32),
            pltpu.VMEM_SHARED((N_PAD,), jnp.float32),
            pltpu.SemaphoreType.DMA,
        ],
    )
    def degree(dst2_hbm, ones_hbm, z_hbm, out_hbm, didx, ones_v, acc, sem):
        c = lax.axis_index("c")
        s = lax.axis_index("s")
        w = s * NC + c
        pltpu.sync_copy(z_hbm.at[pl.ds(s * DROWS, DROWS)],
                        acc.at[pl.ds(s * DROWS, DROWS)])
        pltpu.sync_copy(dst2_hbm.at[w], didx)
        pltpu.sync_copy(ones_hbm, ones_v)
        plsc.subcore_barrier()

        for g in range(NCH_DEG):
            pltpu.async_copy(ones_v, acc.at[didx.at[g]], sem, add=True)
        for g in range(NCH_DEG):
            pltpu.make_async_copy(ones_v, acc.at[didx.at[g]], sem).wait()
        plsc.subcore_barrier()
        pltpu.sync_copy(acc.at[pl.ds(s * DROWS, DROWS)],
                        out_hbm.at[c, pl.ds(s * DROWS, DROWS)])

    return degree


def _make_propagate(D, CP):
    NCH = EW // CP
    NG = NCH // 4
    assert EW % CP == 0 and NCH % 4 == 0 and NG >= 2

    @functools.partial(
        pl.kernel,
        out_type=jax.ShapeDtypeStruct((NC, N_PAD, D), jnp.float32),
        mesh=_mesh(),
        compiler_params=_SC_PARAMS,
        scratch_types=[
            [pltpu.VMEM((2, CP), jnp.int32) for _ in range(4)],
            [pltpu.VMEM((CP, D), jnp.float32) for _ in range(2)],
            pltpu.VMEM_SHARED((N_PAD, D), jnp.float32),
            [pltpu.SemaphoreType.DMA for _ in range(4)],
            [pltpu.SemaphoreType.DMA for _ in range(2)],
            [pltpu.SemaphoreType.DMA for _ in range(2)],
        ],
    )
    def propagate(p_hbm, e4_hbm, z_hbm, out_hbm, I, R, acc, sI, sg, ss):
        c = lax.axis_index("c")
        s = lax.axis_index("s")
        w = s * NC + c
        pltpu.sync_copy(z_hbm.at[pl.ds(s * ROWS, ROWS)],
                        acc.at[pl.ds(s * ROWS, ROWS)])
        plsc.subcore_barrier()

        def idx_load(k, u):
            pltpu.async_copy(e4_hbm.at[w, k], I[u], sI[u])

        def idx_wait(u):
            pltpu.make_async_copy(e4_hbm.at[w, 0], I[u], sI[u]).wait()

        def body(g, do_issue):
            for u in range(4):
                b = u % 2
                pltpu.make_async_copy(p_hbm.at[I[u].at[0]], R[b], sg[b]).wait()
                pltpu.async_copy(R[b], acc.at[I[u].at[1]], ss[b], add=True)
                pltpu.make_async_copy(R[b], acc.at[I[u].at[1]], ss[b]).wait()
                if do_issue:
                    idx_load(4 * g + u + 4, u)
                if do_issue or u < 2:
                    u2 = (u + 2) % 4
                    idx_wait(u2)
                    pltpu.async_copy(p_hbm.at[I[u2].at[0]], R[b], sg[b])

        for u in range(4):
            idx_load(u, u)
        for u in range(2):
            idx_wait(u)
            pltpu.async_copy(p_hbm.at[I[u].at[0]], R[u], sg[u])

        lax.fori_loop(0, NG - 1, lambda g, cy: (body(g, True), cy)[1], 0)
        body(NG - 1, False)

        plsc.subcore_barrier()
        pltpu.sync_copy(acc.at[pl.ds(s * ROWS, ROWS)],
                        out_hbm.at[c, pl.ds(s * ROWS, ROWS)])

    return propagate


BM = 1000


def _dinv_body(deg_ref, mm1_ref, dinv_ref, p1_ref):
    d = deg_ref[0, :] + deg_ref[1, :] + 1.0
    dinv = jax.lax.rsqrt(d)[:, None]
    dinv_ref[...] = dinv
    p1_ref[...] = dinv[:N_NODES] * mm1_ref[...]


def _dinv_scale(deg, mm1):
    return pl.pallas_call(
        _dinv_body,
        out_shape=[
            jax.ShapeDtypeStruct((N_PAD, 1), jnp.float32),
            jax.ShapeDtypeStruct((N_NODES, D_HID), jnp.float32),
        ],
    )(deg, mm1)


def _mm_body(x_ref, w_ref, out_ref):
    out_ref[...] = jnp.dot(x_ref[...], w_ref[...],
                           preferred_element_type=jnp.float32)


def _mm(x, w):
    m, k = x.shape
    n = w.shape[1]
    return pl.pallas_call(
        _mm_body,
        grid=(m // BM,),
        in_specs=[
            pl.BlockSpec((BM, k), lambda i: (i, 0)),
            pl.BlockSpec((k, n), lambda i: (0, 0)),
        ],
        out_specs=pl.BlockSpec((BM, n), lambda i: (i, 0)),
        out_shape=jax.ShapeDtypeStruct((m, n), jnp.float32),
    )(x, w)


def _epilogue_body(nw, d, ac, pc, dinv_ref, a0_ref, a1_ref, p_ref, b_ref, *rest):
    w_refs = rest[:nw]
    out_refs = rest[nw:]
    dinv = dinv_ref[...]
    a0 = a0_ref[0][:, ac * d:(ac + 1) * d]
    a1 = a1_ref[0][:, ac * d:(ac + 1) * d]
    p = p_ref[...][:, pc * d:(pc + 1) * d]
    h = dinv * (a0 + a1 + p) + b_ref[...]
    h = jnp.maximum(h, 0.0)
    if nw == 0:
        out_refs[0][...] = h
    else:
        for w_ref, o_ref in zip(w_refs, out_refs):
            t = jnp.dot(h, w_ref[...], preferred_element_type=jnp.float32)
            o_ref[...] = dinv * t


def _epilogue(dinv, a, p, b, ws, d, ac=0, pc=0):
    m = N_NODES
    nw = len(ws)
    outs = ([jax.ShapeDtypeStruct((m, d), jnp.float32)] if nw == 0 else
            [jax.ShapeDtypeStruct((m, w.shape[1]), jnp.float32) for w in ws])
    w_specs = [pl.BlockSpec(w.shape, lambda i: (0, 0)) for w in ws]
    out_specs = ([pl.BlockSpec((BM, d), lambda i: (i, 0))] if nw == 0 else
                 [pl.BlockSpec((BM, w.shape[1]), lambda i: (i, 0)) for w in ws])
    wa = a.shape[2]
    wp = p.shape[1]
    res = pl.pallas_call(
        functools.partial(_epilogue_body, nw, d, ac, pc),
        grid=(m // BM,),
        in_specs=[
            pl.BlockSpec((BM, 1), lambda i: (i, 0)),
            pl.BlockSpec((1, BM, wa), lambda i: (0, i, 0)),
            pl.BlockSpec((1, BM, wa), lambda i: (1, i, 0)),
            pl.BlockSpec((BM, wp), lambda i: (i, 0)),
            pl.BlockSpec((1, d), lambda i: (0, 0)),
            *w_specs,
        ],
        out_specs=out_specs,
        out_shape=outs,
    )(dinv, a, a, p, b, *ws)
    return res


BN = 1024


def _sst_body(sr_ref, sc_ref, out_ref):
    out_ref[...] = lax.dot_general(
        sr_ref[...], sc_ref[...],
        dimension_numbers=(((1,), (1,)), ((), ())),
        preferred_element_type=jnp.float32,
    )


def _s_st(s):
    m = s.shape[0]
    d = s.shape[1]
    return pl.pallas_call(
        _sst_body,
        grid=(m // BM, pl.cdiv(m, BN)),
        in_specs=[
            pl.BlockSpec((BM, d), lambda i, j: (i, 0)),
            pl.BlockSpec((BN, d), lambda i, j: (j, 0)),
        ],
        out_specs=pl.BlockSpec((BM, BN), lambda i, j: (i, j)),
        out_shape=jax.ShapeDtypeStruct((m, m), jnp.float32),
    )(s, s)


def kernel(x, edge_index, label, prior_labels, W1, b1, W2, b2, W3, b3, W4, b4, W5, b5):
    ei = edge_index.astype(jnp.int32)
    C64, C128 = 500, 125
    e64 = ei.reshape(2, NW, EW // C64, C64).transpose(1, 2, 0, 3)
    e128 = ei.reshape(2, NW, EW // C128, C128).transpose(1, 2, 0, 3)
    dstdeg = ei[1].reshape(NW, NCH_DEG, C_DEG)

    z1 = jnp.zeros((N_PAD,), jnp.float32)
    z64 = jnp.zeros((N_PAD, D_HID), jnp.float32)
    z128 = jnp.zeros((N_PAD, D_FEAT), jnp.float32)

    ones_blk = jnp.ones((C_DEG,), jnp.float32)
    deg = _make_degree()(dstdeg, ones_blk, z1)
    mm1 = _mm(x, W1)
    dinv, p1 = _dinv_scale(deg, mm1)

    prop64 = _make_propagate(D_HID, C64)
    prop128 = _make_propagate(D_FEAT, C128)

    b1r = b1.reshape(1, -1)
    b2r = b2.reshape(1, -1)
    b3r = b3.reshape(1, -1)
    b4r = b4.reshape(1, -1)
    b5r = b5.reshape(1, -1)
    W35 = jnp.concatenate([W3, W5], axis=1)

    a1 = prop64(p1, e64, z64)
    (p2,) = _epilogue(dinv, a1, p1, b1r, [W2], D_HID)
    a2 = prop64(p2, e64, z64)
    (t35,) = _epilogue(dinv, a2, p2, b2r, [W35], D_HID)

    a35 = prop128(t35, e128, z128)

    (s,) = _epilogue(dinv, a35, t35, b5r, [], D_HID, ac=1, pc=1)
    struct = _s_st(s)

    (p4,) = _epilogue(dinv, a35, t35, b3r, [W4], D_HID, ac=0, pc=0)
    a4 = prop128(p4, e128, z128)
    (x_hat,) = _epilogue(dinv, a4, p4, b4r, [], D_FEAT)

    return (struct, x_hat, edge_index)


# speedup vs baseline: 25.8783x; 1.0000x over previous
import functools

import jax
import jax.numpy as jnp
from jax import lax
from jax.experimental import pallas as pl
from jax.experimental.pallas import tpu as pltpu
from jax.experimental.pallas import tpu_sc as plsc

N_NODES = 10000
N_EDGES = 320000
D_FEAT = 128
D_HID = 64

NC = 2
NS = 16
NW = NC * NS
EW = N_EDGES // NW
N_PAD = 10240
ROWS = N_PAD // NS
DROWS = N_PAD // NS

_SC_PARAMS = pltpu.CompilerParams(use_tc_tiling_on_sc=False)


def _mesh():
    return plsc.VectorSubcoreMesh(core_axis_name="c", subcore_axis_name="s")


C_DEG = 625
NCH_DEG = EW // C_DEG


def _make_degree():
    @functools.partial(
        pl.kernel,
        out_type=jax.ShapeDtypeStruct((NC, N_PAD), jnp.float32),
        mesh=_mesh(),
        compiler_params=_SC_PARAMS,
        scratch_types=[
            pltpu.VMEM((NCH_DEG, C_DEG), jnp.int32),
            pltpu.VMEM((C_DEG,), jnp.float32),
            pltpu.VMEM_SHARED((N_PAD,), jnp.float32),
            pltpu.SemaphoreType.DMA,
        ],
    )
    def degree(dst2_hbm, ones_hbm, z_hbm, out_hbm, didx, ones_v, acc, sem):
        c = lax.axis_index("c")
        s = lax.axis_index("s")
        w = s * NC + c
        pltpu.sync_copy(z_hbm.at[pl.ds(s * DROWS, DROWS)],
                        acc.at[pl.ds(s * DROWS, DROWS)])
        pltpu.sync_copy(dst2_hbm.at[w], didx)
        pltpu.sync_copy(ones_hbm, ones_v)
        plsc.subcore_barrier()

        for g in range(NCH_DEG):
            pltpu.async_copy(ones_v, acc.at[didx.at[g]], sem, add=True)
        for g in range(NCH_DEG):
            pltpu.make_async_copy(ones_v, acc.at[didx.at[g]], sem).wait()
        plsc.subcore_barrier()
        pltpu.sync_copy(acc.at[pl.ds(s * DROWS, DROWS)],
                        out_hbm.at[c, pl.ds(s * DROWS, DROWS)])

    return degree


def _make_propagate(D, CP):
    NCH = EW // CP
    NG = NCH // 4
    assert EW % CP == 0 and NCH % 4 == 0 and NG >= 2

    @functools.partial(
        pl.kernel,
        out_type=jax.ShapeDtypeStruct((NC, N_PAD, D), jnp.float32),
        mesh=_mesh(),
        compiler_params=_SC_PARAMS,
        scratch_types=[
            [pltpu.VMEM((2, CP), jnp.int32) for _ in range(4)],
            [pltpu.VMEM((CP, D), jnp.float32) for _ in range(2)],
            pltpu.VMEM_SHARED((N_PAD, D), jnp.float32),
            [pltpu.SemaphoreType.DMA for _ in range(4)],
            [pltpu.SemaphoreType.DMA for _ in range(2)],
            [pltpu.SemaphoreType.DMA for _ in range(2)],
        ],
    )
    def propagate(p_hbm, e4_hbm, z_hbm, out_hbm, I, R, acc, sI, sg, ss):
        c = lax.axis_index("c")
        s = lax.axis_index("s")
        w = s * NC + c
        pltpu.sync_copy(z_hbm.at[pl.ds(s * ROWS, ROWS)],
                        acc.at[pl.ds(s * ROWS, ROWS)])
        plsc.subcore_barrier()

        def idx_load(k, u):
            pltpu.async_copy(e4_hbm.at[w, k], I[u], sI[u])

        def idx_wait(u):
            pltpu.make_async_copy(e4_hbm.at[w, 0], I[u], sI[u]).wait()

        def body(g, do_issue):
            for u in range(4):
                b = u % 2
                pltpu.make_async_copy(p_hbm.at[I[u].at[0]], R[b], sg[b]).wait()
                pltpu.async_copy(R[b], acc.at[I[u].at[1]], ss[b], add=True)
                pltpu.make_async_copy(R[b], acc.at[I[u].at[1]], ss[b]).wait()
                if do_issue:
                    idx_load(4 * g + u + 4, u)
                if do_issue or u < 2:
                    u2 = (u + 2) % 4
                    idx_wait(u2)
                    pltpu.async_copy(p_hbm.at[I[u2].at[0]], R[b], sg[b])

        for u in range(4):
            idx_load(u, u)
        for u in range(2):
            idx_wait(u)
            pltpu.async_copy(p_hbm.at[I[u].at[0]], R[u], sg[u])

        lax.fori_loop(0, NG - 1, lambda g, cy: (body(g, True), cy)[1], 0)
        body(NG - 1, False)

        plsc.subcore_barrier()
        pltpu.sync_copy(acc.at[pl.ds(s * ROWS, ROWS)],
                        out_hbm.at[c, pl.ds(s * ROWS, ROWS)])

    return propagate


def _make_propagate4(D, CP):
    NCH = EW // CP
    assert EW % CP == 0 and NCH % 4 == 0 and NCH >= 8

    @functools.partial(
        pl.kernel,
        out_type=jax.ShapeDtypeStruct((NC, N_PAD, D), jnp.float32),
        mesh=_mesh(),
        compiler_params=_SC_PARAMS,
        scratch_types=[
            pltpu.VMEM((NCH, 2, CP), jnp.int32),
            [pltpu.VMEM((CP, D), jnp.float32) for _ in range(4)],
            pltpu.VMEM_SHARED((N_PAD, D), jnp.float32),
            [pltpu.SemaphoreType.DMA for _ in range(4)],
            [pltpu.SemaphoreType.DMA for _ in range(4)],
        ],
    )
    def propagate(p_hbm, e4_hbm, z_hbm, out_hbm, E, R, acc, sg, ss):
        c = lax.axis_index("c")
        s = lax.axis_index("s")
        w = s * NC + c
        pltpu.sync_copy(z_hbm.at[pl.ds(s * ROWS, ROWS)],
                        acc.at[pl.ds(s * ROWS, ROWS)])
        pltpu.sync_copy(e4_hbm.at[w], E)
        plsc.subcore_barrier()

        def gath(k, b):
            pltpu.async_copy(p_hbm.at[E.at[k, 0]], R[b], sg[b])

        def wait_gath(k, b):
            pltpu.make_async_copy(p_hbm.at[E.at[k, 0]], R[b], sg[b]).wait()

        def scat(k, b):
            pltpu.async_copy(R[b], acc.at[E.at[k, 1]], ss[b], add=True)

        def wait_scat(k, b):
            pltpu.make_async_copy(R[b], acc.at[E.at[k, 1]], ss[b]).wait()

        gath(0, 0)
        gath(1, 1)
        wait_gath(0, 0)
        scat(0, 0)
        gath(2, 2)
        wait_gath(1, 1)
        scat(1, 1)
        gath(3, 3)

        def body(g, cy):
            j0 = 4 * g + 2
            for u in range(4):
                b = (2 + u) % 4
                j = j0 + u
                wait_gath(j, b)
                scat(j, b)
                wait_scat(j - 2, (b + 2) % 4)
                gath(j + 2, (b + 2) % 4)
            return cy

        lax.fori_loop(0, (NCH - 4) // 4, body, 0)

        for u in range(2):
            j = NCH - 2 + u
            b = j % 4
            wait_gath(j, b)
            scat(j, b)
            wait_scat(j - 2, (b + 2) % 4)
        wait_scat(NCH - 2, (NCH - 2) % 4)
        wait_scat(NCH - 1, (NCH - 1) % 4)

        plsc.subcore_barrier()
        pltpu.sync_copy(acc.at[pl.ds(s * ROWS, ROWS)],
                        out_hbm.at[c, pl.ds(s * ROWS, ROWS)])

    return propagate


BM = 1000


def _dinv_body(deg_ref, mm1_ref, dinv_ref, p1_ref):
    d = deg_ref[0, :] + deg_ref[1, :] + 1.0
    dinv = jax.lax.rsqrt(d)[:, None]
    dinv_ref[...] = dinv
    p1_ref[...] = dinv[:N_NODES] * mm1_ref[...]


def _dinv_scale(deg, mm1):
    return pl.pallas_call(
        _dinv_body,
        out_shape=[
            jax.ShapeDtypeStruct((N_PAD, 1), jnp.float32),
            jax.ShapeDtypeStruct((N_NODES, D_HID), jnp.float32),
        ],
    )(deg, mm1)


def _mm_body(x_ref, w_ref, out_ref):
    out_ref[...] = jnp.dot(x_ref[...], w_ref[...],
                           preferred_element_type=jnp.float32)


def _mm(x, w):
    m, k = x.shape
    n = w.shape[1]
    return pl.pallas_call(
        _mm_body,
        grid=(m // BM,),
        in_specs=[
            pl.BlockSpec((BM, k), lambda i: (i, 0)),
            pl.BlockSpec((k, n), lambda i: (0, 0)),
        ],
        out_specs=pl.BlockSpec((BM, n), lambda i: (i, 0)),
        out_shape=jax.ShapeDtypeStruct((m, n), jnp.float32),
    )(x, w)


def _epilogue_body(nw, d, ac, pc, dinv_ref, a0_ref, a1_ref, p_ref, b_ref, *rest):
    w_refs = rest[:nw]
    out_refs = rest[nw:]
    dinv = dinv_ref[...]
    a0 = a0_ref[0][:, ac * d:(ac + 1) * d]
    a1 = a1_ref[0][:, ac * d:(ac + 1) * d]
    p = p_ref[...][:, pc * d:(pc + 1) * d]
    h = dinv * (a0 + a1 + p) + b_ref[...]
    h = jnp.maximum(h, 0.0)
    if nw == 0:
        out_refs[0][...] = h
    else:
        for w_ref, o_ref in zip(w_refs, out_refs):
            t = jnp.dot(h, w_ref[...], preferred_element_type=jnp.float32)
            o_ref[...] = dinv * t


def _epilogue(dinv, a, p, b, ws, d, ac=0, pc=0):
    m = N_NODES
    nw = len(ws)
    outs = ([jax.ShapeDtypeStruct((m, d), jnp.float32)] if nw == 0 else
            [jax.ShapeDtypeStruct((m, w.shape[1]), jnp.float32) for w in ws])
    w_specs = [pl.BlockSpec(w.shape, lambda i: (0, 0)) for w in ws]
    out_specs = ([pl.BlockSpec((BM, d), lambda i: (i, 0))] if nw == 0 else
                 [pl.BlockSpec((BM, w.shape[1]), lambda i: (i, 0)) for w in ws])
    wa = a.shape[2]
    wp = p.shape[1]
    res = pl.pallas_call(
        functools.partial(_epilogue_body, nw, d, ac, pc),
        grid=(m // BM,),
        in_specs=[
            pl.BlockSpec((BM, 1), lambda i: (i, 0)),
            pl.BlockSpec((1, BM, wa), lambda i: (0, i, 0)),
            pl.BlockSpec((1, BM, wa), lambda i: (1, i, 0)),
            pl.BlockSpec((BM, wp), lambda i: (i, 0)),
            pl.BlockSpec((1, d), lambda i: (0, 0)),
            *w_specs,
        ],
        out_specs=out_specs,
        out_shape=outs,
    )(dinv, a, a, p, b, *ws)
    return res


BN = 1024


def _sst_body(sr_ref, sc_ref, out_ref):
    out_ref[...] = lax.dot_general(
        sr_ref[...], sc_ref[...],
        dimension_numbers=(((1,), (1,)), ((), ())),
        preferred_element_type=jnp.float32,
    )


def _s_st(s):
    m = s.shape[0]
    d = s.shape[1]
    return pl.pallas_call(
        _sst_body,
        grid=(m // BM, pl.cdiv(m, BN)),
        in_specs=[
            pl.BlockSpec((BM, d), lambda i, j: (i, 0)),
            pl.BlockSpec((BN, d), lambda i, j: (j, 0)),
        ],
        out_specs=pl.BlockSpec((BM, BN), lambda i, j: (i, j)),
        out_shape=jax.ShapeDtypeStruct((m, m), jnp.float32),
    )(s, s)


def kernel(x, edge_index, label, prior_labels, W1, b1, W2, b2, W3, b3, W4, b4, W5, b5):
    ei = edge_index.astype(jnp.int32)
    C64, C128 = 250, 125
    e64 = ei.reshape(2, NW, EW // C64, C64).transpose(1, 2, 0, 3)
    e128 = ei.reshape(2, NW, EW // C128, C128).transpose(1, 2, 0, 3)
    dstdeg = ei[1].reshape(NW, NCH_DEG, C_DEG)

    z1 = jnp.zeros((N_PAD,), jnp.float32)
    z64 = jnp.zeros((N_PAD, D_HID), jnp.float32)
    z128 = jnp.zeros((N_PAD, D_FEAT), jnp.float32)

    ones_blk = jnp.ones((C_DEG,), jnp.float32)
    deg = _make_degree()(dstdeg, ones_blk, z1)
    mm1 = _mm(x, W1)
    dinv, p1 = _dinv_scale(deg, mm1)

    prop64 = _make_propagate4(D_HID, C64)
    prop128 = _make_propagate(D_FEAT, C128)

    b1r = b1.reshape(1, -1)
    b2r = b2.reshape(1, -1)
    b3r = b3.reshape(1, -1)
    b4r = b4.reshape(1, -1)
    b5r = b5.reshape(1, -1)
    W35 = jnp.concatenate([W3, W5], axis=1)

    a1 = prop64(p1, e64, z64)
    (p2,) = _epilogue(dinv, a1, p1, b1r, [W2], D_HID)
    a2 = prop64(p2, e64, z64)
    (t35,) = _epilogue(dinv, a2, p2, b2r, [W35], D_HID)

    a35 = prop128(t35, e128, z128)

    (s,) = _epilogue(dinv, a35, t35, b5r, [], D_HID, ac=1, pc=1)
    struct = _s_st(s)

    (p4,) = _epilogue(dinv, a35, t35, b3r, [W4], D_HID, ac=0, pc=0)
    a4 = prop128(p4, e128, z128)
    (x_hat,) = _epilogue(dinv, a4, p4, b4r, [], D_FEAT)

    return (struct, x_hat, edge_index)


# speedup vs baseline: 25.9754x; 1.0038x over previous
import functools

import jax
import jax.numpy as jnp
from jax import lax
from jax.experimental import pallas as pl
from jax.experimental.pallas import tpu as pltpu
from jax.experimental.pallas import tpu_sc as plsc

N_NODES = 10000
N_EDGES = 320000
D_FEAT = 128
D_HID = 64

NC = 2
NS = 16
NW = NC * NS
EW = N_EDGES // NW
N_PAD = 10240
ROWS = N_PAD // NS
DROWS = N_PAD // NS

_SC_PARAMS = pltpu.CompilerParams(use_tc_tiling_on_sc=False)


def _mesh():
    return plsc.VectorSubcoreMesh(core_axis_name="c", subcore_axis_name="s")


C_DEG = 625
NCH_DEG = EW // C_DEG


def _make_degree():
    @functools.partial(
        pl.kernel,
        out_type=jax.ShapeDtypeStruct((NC, N_PAD), jnp.float32),
        mesh=_mesh(),
        compiler_params=_SC_PARAMS,
        scratch_types=[
            pltpu.VMEM((NCH_DEG, C_DEG), jnp.int32),
            pltpu.VMEM((C_DEG,), jnp.float32),
            pltpu.VMEM_SHARED((N_PAD,), jnp.float32),
            pltpu.SemaphoreType.DMA,
        ],
    )
    def degree(dst2_hbm, ones_hbm, z_hbm, out_hbm, didx, ones_v, acc, sem):
        c = lax.axis_index("c")
        s = lax.axis_index("s")
        w = s * NC + c
        pltpu.sync_copy(z_hbm.at[pl.ds(s * DROWS, DROWS)],
                        acc.at[pl.ds(s * DROWS, DROWS)])
        pltpu.sync_copy(dst2_hbm.at[w], didx)
        pltpu.sync_copy(ones_hbm, ones_v)
        plsc.subcore_barrier()

        for g in range(NCH_DEG):
            pltpu.async_copy(ones_v, acc.at[didx.at[g]], sem, add=True)
        for g in range(NCH_DEG):
            pltpu.make_async_copy(ones_v, acc.at[didx.at[g]], sem).wait()
        plsc.subcore_barrier()
        pltpu.sync_copy(acc.at[pl.ds(s * DROWS, DROWS)],
                        out_hbm.at[c, pl.ds(s * DROWS, DROWS)])

    return degree


def _make_propagate(D, CP):
    NCH = EW // CP
    NG = NCH // 4
    assert EW % CP == 0 and NCH % 4 == 0 and NG >= 2

    @functools.partial(
        pl.kernel,
        out_type=jax.ShapeDtypeStruct((NC, N_PAD, D), jnp.float32),
        mesh=_mesh(),
        compiler_params=_SC_PARAMS,
        scratch_types=[
            [pltpu.VMEM((2, CP), jnp.int32) for _ in range(4)],
            [pltpu.VMEM((CP, D), jnp.float32) for _ in range(2)],
            pltpu.VMEM_SHARED((N_PAD, D), jnp.float32),
            [pltpu.SemaphoreType.DMA for _ in range(4)],
            [pltpu.SemaphoreType.DMA for _ in range(2)],
            [pltpu.SemaphoreType.DMA for _ in range(2)],
        ],
    )
    def propagate(p_hbm, e4_hbm, z_hbm, out_hbm, I, R, acc, sI, sg, ss):
        c = lax.axis_index("c")
        s = lax.axis_index("s")
        w = s * NC + c
        pltpu.sync_copy(z_hbm.at[pl.ds(s * ROWS, ROWS)],
                        acc.at[pl.ds(s * ROWS, ROWS)])
        plsc.subcore_barrier()

        def idx_load(k, u):
            pltpu.async_copy(e4_hbm.at[w, k], I[u], sI[u])

        def idx_wait(u):
            pltpu.make_async_copy(e4_hbm.at[w, 0], I[u], sI[u]).wait()

        def body(g, do_issue):
            for u in range(4):
                b = u % 2
                pltpu.make_async_copy(p_hbm.at[I[u].at[0]], R[b], sg[b]).wait()
                pltpu.async_copy(R[b], acc.at[I[u].at[1]], ss[b], add=True)
                pltpu.make_async_copy(R[b], acc.at[I[u].at[1]], ss[b]).wait()
                if do_issue:
                    idx_load(4 * g + u + 4, u)
                if do_issue or u < 2:
                    u2 = (u + 2) % 4
                    idx_wait(u2)
                    pltpu.async_copy(p_hbm.at[I[u2].at[0]], R[b], sg[b])

        for u in range(4):
            idx_load(u, u)
        for u in range(2):
            idx_wait(u)
            pltpu.async_copy(p_hbm.at[I[u].at[0]], R[u], sg[u])

        lax.fori_loop(0, NG - 1, lambda g, cy: (body(g, True), cy)[1], 0)
        body(NG - 1, False)

        plsc.subcore_barrier()
        pltpu.sync_copy(acc.at[pl.ds(s * ROWS, ROWS)],
                        out_hbm.at[c, pl.ds(s * ROWS, ROWS)])

    return propagate


def _make_propagate4(D, CP):
    NCH = EW // CP
    assert EW % CP == 0 and NCH % 4 == 0 and NCH >= 8

    @functools.partial(
        pl.kernel,
        out_type=jax.ShapeDtypeStruct((NC, N_PAD, D), jnp.float32),
        mesh=_mesh(),
        compiler_params=_SC_PARAMS,
        scratch_types=[
            pltpu.VMEM((NCH, 2, CP), jnp.int32),
            [pltpu.VMEM((CP, D), jnp.float32) for _ in range(4)],
            pltpu.VMEM_SHARED((N_PAD, D), jnp.float32),
            [pltpu.SemaphoreType.DMA for _ in range(4)],
            [pltpu.SemaphoreType.DMA for _ in range(4)],
        ],
    )
    def propagate(p_hbm, e4_hbm, z_hbm, out_hbm, E, R, acc, sg, ss):
        c = lax.axis_index("c")
        s = lax.axis_index("s")
        w = s * NC + c
        pltpu.sync_copy(z_hbm.at[pl.ds(s * ROWS, ROWS)],
                        acc.at[pl.ds(s * ROWS, ROWS)])
        pltpu.sync_copy(e4_hbm.at[w], E)
        plsc.subcore_barrier()

        def gath(k, b):
            pltpu.async_copy(p_hbm.at[E.at[k, 0]], R[b], sg[b])

        def wait_gath(k, b):
            pltpu.make_async_copy(p_hbm.at[E.at[k, 0]], R[b], sg[b]).wait()

        def scat(k, b):
            pltpu.async_copy(R[b], acc.at[E.at[k, 1]], ss[b], add=True)

        def wait_scat(k, b):
            pltpu.make_async_copy(R[b], acc.at[E.at[k, 1]], ss[b]).wait()

        gath(0, 0)
        gath(1, 1)
        wait_gath(0, 0)
        scat(0, 0)
        gath(2, 2)
        wait_gath(1, 1)
        scat(1, 1)
        gath(3, 3)

        def body(g, cy):
            j0 = 4 * g + 2
            for u in range(4):
                b = (2 + u) % 4
                j = j0 + u
                wait_gath(j, b)
                scat(j, b)
                wait_scat(j - 2, (b + 2) % 4)
                gath(j + 2, (b + 2) % 4)
            return cy

        lax.fori_loop(0, (NCH - 4) // 4, body, 0)

        for u in range(2):
            j = NCH - 2 + u
            b = j % 4
            wait_gath(j, b)
            scat(j, b)
            wait_scat(j - 2, (b + 2) % 4)
        wait_scat(NCH - 2, (NCH - 2) % 4)
        wait_scat(NCH - 1, (NCH - 1) % 4)

        plsc.subcore_barrier()
        pltpu.sync_copy(acc.at[pl.ds(s * ROWS, ROWS)],
                        out_hbm.at[c, pl.ds(s * ROWS, ROWS)])

    return propagate


BM = 1000


def _dinv_body(deg_ref, mm1_ref, dinv_ref, p1_ref):
    d = deg_ref[0, :] + deg_ref[1, :] + 1.0
    dinv = jax.lax.rsqrt(d)[:, None]
    dinv_ref[...] = dinv
    p1_ref[...] = dinv[:N_NODES] * mm1_ref[...]


def _dinv_scale(deg, mm1):
    return pl.pallas_call(
        _dinv_body,
        out_shape=[
            jax.ShapeDtypeStruct((N_PAD, 1), jnp.float32),
            jax.ShapeDtypeStruct((N_NODES, D_HID), jnp.float32),
        ],
    )(deg, mm1)


def _mm_body(x_ref, w_ref, out_ref):
    out_ref[...] = jnp.dot(x_ref[...], w_ref[...],
                           preferred_element_type=jnp.float32)


def _mm(x, w):
    m, k = x.shape
    n = w.shape[1]
    return pl.pallas_call(
        _mm_body,
        grid=(m // BM,),
        in_specs=[
            pl.BlockSpec((BM, k), lambda i: (i, 0)),
            pl.BlockSpec((k, n), lambda i: (0, 0)),
        ],
        out_specs=pl.BlockSpec((BM, n), lambda i: (i, 0)),
        out_shape=jax.ShapeDtypeStruct((m, n), jnp.float32),
    )(x, w)


def _epilogue_body(nw, d, ac, pc, dinv_ref, a0_ref, a1_ref, p_ref, b_ref, *rest):
    w_refs = rest[:nw]
    out_refs = rest[nw:]
    dinv = dinv_ref[...]
    a0 = a0_ref[0][:, ac * d:(ac + 1) * d]
    a1 = a1_ref[0][:, ac * d:(ac + 1) * d]
    p = p_ref[...][:, pc * d:(pc + 1) * d]
    h = dinv * (a0 + a1 + p) + b_ref[...]
    h = jnp.maximum(h, 0.0)
    if nw == 0:
        out_refs[0][...] = h
    else:
        for w_ref, o_ref in zip(w_refs, out_refs):
            t = jnp.dot(h, w_ref[...], preferred_element_type=jnp.float32)
            o_ref[...] = dinv * t


def _epilogue(dinv, a, p, b, ws, d, ac=0, pc=0):
    m = N_NODES
    nw = len(ws)
    outs = ([jax.ShapeDtypeStruct((m, d), jnp.float32)] if nw == 0 else
            [jax.ShapeDtypeStruct((m, w.shape[1]), jnp.float32) for w in ws])
    w_specs = [pl.BlockSpec(w.shape, lambda i: (0, 0)) for w in ws]
    out_specs = ([pl.BlockSpec((BM, d), lambda i: (i, 0))] if nw == 0 else
                 [pl.BlockSpec((BM, w.shape[1]), lambda i: (i, 0)) for w in ws])
    wa = a.shape[2]
    wp = p.shape[1]
    res = pl.pallas_call(
        functools.partial(_epilogue_body, nw, d, ac, pc),
        grid=(m // BM,),
        in_specs=[
            pl.BlockSpec((BM, 1), lambda i: (i, 0)),
            pl.BlockSpec((1, BM, wa), lambda i: (0, i, 0)),
            pl.BlockSpec((1, BM, wa), lambda i: (1, i, 0)),
            pl.BlockSpec((BM, wp), lambda i: (i, 0)),
            pl.BlockSpec((1, d), lambda i: (0, 0)),
            *w_specs,
        ],
        out_specs=out_specs,
        out_shape=outs,
    )(dinv, a, a, p, b, *ws)
    return res


BN = 2048


def _sst_body(sr_ref, sc_ref, out_ref):
    out_ref[...] = lax.dot_general(
        sr_ref[...], sc_ref[...],
        dimension_numbers=(((1,), (1,)), ((), ())),
        preferred_element_type=jnp.float32,
    )


def _s_st(s):
    m = s.shape[0]
    d = s.shape[1]
    return pl.pallas_call(
        _sst_body,
        grid=(m // BM, pl.cdiv(m, BN)),
        in_specs=[
            pl.BlockSpec((BM, d), lambda i, j: (i, 0)),
            pl.BlockSpec((BN, d), lambda i, j: (j, 0)),
        ],
        out_specs=pl.BlockSpec((BM, BN), lambda i, j: (i, j)),
        out_shape=jax.ShapeDtypeStruct((m, m), jnp.float32),
    )(s, s)


def kernel(x, edge_index, label, prior_labels, W1, b1, W2, b2, W3, b3, W4, b4, W5, b5):
    ei = edge_index.astype(jnp.int32)
    C64, C128 = 125, 125
    e64 = ei.reshape(2, NW, EW // C64, C64).transpose(1, 2, 0, 3)
    e128 = e64
    dstdeg = ei[1].reshape(NW, NCH_DEG, C_DEG)

    z1 = jnp.zeros((N_PAD,), jnp.float32)
    z64 = jnp.zeros((N_PAD, D_HID), jnp.float32)
    z128 = jnp.zeros((N_PAD, D_FEAT), jnp.float32)

    ones_blk = jnp.ones((C_DEG,), jnp.float32)
    deg = _make_degree()(dstdeg, ones_blk, z1)
    mm1 = _mm(x, W1)
    dinv, p1 = _dinv_scale(deg, mm1)

    prop64 = _make_propagate4(D_HID, C64)
    prop128 = _make_propagate(D_FEAT, C128)

    b1r = b1.reshape(1, -1)
    b2r = b2.reshape(1, -1)
    b3r = b3.reshape(1, -1)
    b4r = b4.reshape(1, -1)
    b5r = b5.reshape(1, -1)
    W35 = jnp.concatenate([W3, W5], axis=1)

    a1 = prop64(p1, e64, z64)
    (p2,) = _epilogue(dinv, a1, p1, b1r, [W2], D_HID)
    a2 = prop64(p2, e64, z64)
    (t35,) = _epilogue(dinv, a2, p2, b2r, [W35], D_HID)

    a35 = prop128(t35, e128, z128)

    (s,) = _epilogue(dinv, a35, t35, b5r, [], D_HID, ac=1, pc=1)
    struct = _s_st(s)

    (p4,) = _epilogue(dinv, a35, t35, b3r, [W4], D_HID, ac=0, pc=0)
    a4 = prop128(p4, e128, z128)
    (x_hat,) = _epilogue(dinv, a4, p4, b4r, [], D_FEAT)

    return (struct, x_hat, edge_index)


# speedup vs baseline: 28.6349x; 1.1024x over previous
import functools

import jax
import jax.numpy as jnp
from jax import lax
from jax.experimental import pallas as pl
from jax.experimental.pallas import tpu as pltpu
from jax.experimental.pallas import tpu_sc as plsc

N_NODES = 10000
N_EDGES = 320000
D_FEAT = 128
D_HID = 64

NC = 2
NS = 16
NW = NC * NS
EW = N_EDGES // NW
N_PAD = 10240
ROWS = N_PAD // NS
DROWS = N_PAD // NS

_SC_PARAMS = pltpu.CompilerParams(use_tc_tiling_on_sc=False)


def _mesh():
    return plsc.VectorSubcoreMesh(core_axis_name="c", subcore_axis_name="s")


C_DEG = 625
NCH_DEG = EW // C_DEG


def _make_degree():
    @functools.partial(
        pl.kernel,
        out_type=jax.ShapeDtypeStruct((NC, N_PAD), jnp.float32),
        mesh=_mesh(),
        compiler_params=_SC_PARAMS,
        scratch_types=[
            pltpu.VMEM((NCH_DEG, C_DEG), jnp.int32),
            pltpu.VMEM((C_DEG,), jnp.float32),
            pltpu.VMEM_SHARED((N_PAD,), jnp.float32),
            pltpu.SemaphoreType.DMA,
        ],
    )
    def degree(dst2_hbm, ones_hbm, z_hbm, out_hbm, didx, ones_v, acc, sem):
        c = lax.axis_index("c")
        s = lax.axis_index("s")
        w = s * NC + c
        pltpu.sync_copy(z_hbm.at[pl.ds(s * DROWS, DROWS)],
                        acc.at[pl.ds(s * DROWS, DROWS)])
        pltpu.sync_copy(dst2_hbm.at[w], didx)
        pltpu.sync_copy(ones_hbm, ones_v)
        plsc.subcore_barrier()

        for g in range(NCH_DEG):
            pltpu.async_copy(ones_v, acc.at[didx.at[g]], sem, add=True)
        for g in range(NCH_DEG):
            pltpu.make_async_copy(ones_v, acc.at[didx.at[g]], sem).wait()
        plsc.subcore_barrier()
        pltpu.sync_copy(acc.at[pl.ds(s * DROWS, DROWS)],
                        out_hbm.at[c, pl.ds(s * DROWS, DROWS)])

    return degree


def _make_propagate4(D, CP):
    NCH = EW // CP
    assert EW % CP == 0 and NCH % 4 == 0 and NCH >= 8

    @functools.partial(
        pl.kernel,
        out_type=jax.ShapeDtypeStruct((NC, N_PAD, D), jnp.float32),
        mesh=_mesh(),
        compiler_params=_SC_PARAMS,
        scratch_types=[
            pltpu.VMEM((NCH, 2, CP), jnp.int32),
            [pltpu.VMEM((CP, D), jnp.float32) for _ in range(4)],
            pltpu.VMEM_SHARED((N_PAD, D), jnp.float32),
            [pltpu.SemaphoreType.DMA for _ in range(4)],
            [pltpu.SemaphoreType.DMA for _ in range(4)],
        ],
    )
    def propagate(p_hbm, e4_hbm, z_hbm, out_hbm, E, R, acc, sg, ss):
        c = lax.axis_index("c")
        s = lax.axis_index("s")
        w = s * NC + c
        pltpu.sync_copy(z_hbm.at[pl.ds(s * ROWS, ROWS)],
                        acc.at[pl.ds(s * ROWS, ROWS)])
        pltpu.sync_copy(e4_hbm.at[w], E)
        plsc.subcore_barrier()

        def gath(k, b):
            pltpu.async_copy(p_hbm.at[E.at[k, 0]], R[b], sg[b])

        def wait_gath(k, b):
            pltpu.make_async_copy(p_hbm.at[E.at[k, 0]], R[b], sg[b]).wait()

        def scat(k, b):
            pltpu.async_copy(R[b], acc.at[E.at[k, 1]], ss[b], add=True)

        def wait_scat(k, b):
            pltpu.make_async_copy(R[b], acc.at[E.at[k, 1]], ss[b]).wait()

        gath(0, 0)
        gath(1, 1)
        wait_gath(0, 0)
        scat(0, 0)
        gath(2, 2)
        wait_gath(1, 1)
        scat(1, 1)
        gath(3, 3)

        def body(g, cy):
            j0 = 4 * g + 2
            for u in range(4):
                b = (2 + u) % 4
                j = j0 + u
                wait_gath(j, b)
                scat(j, b)
                wait_scat(j - 2, (b + 2) % 4)
                gath(j + 2, (b + 2) % 4)
            return cy

        lax.fori_loop(0, (NCH - 4) // 4, body, 0)

        for u in range(2):
            j = NCH - 2 + u
            b = j % 4
            wait_gath(j, b)
            scat(j, b)
            wait_scat(j - 2, (b + 2) % 4)
        wait_scat(NCH - 2, (NCH - 2) % 4)
        wait_scat(NCH - 1, (NCH - 1) % 4)

        plsc.subcore_barrier()
        pltpu.sync_copy(acc.at[pl.ds(s * ROWS, ROWS)],
                        out_hbm.at[c, pl.ds(s * ROWS, ROWS)])

    return propagate


BM = 1000


def _dinv_body(deg_ref, mm1_ref, dinv_ref, p1_ref):
    d = deg_ref[0, :] + deg_ref[1, :] + 1.0
    dinv = jax.lax.rsqrt(d)[:, None]
    dinv_ref[...] = dinv
    p1_ref[...] = dinv[:N_NODES] * mm1_ref[...]


def _dinv_scale(deg, mm1):
    return pl.pallas_call(
        _dinv_body,
        out_shape=[
            jax.ShapeDtypeStruct((N_PAD, 1), jnp.float32),
            jax.ShapeDtypeStruct((N_NODES, D_HID), jnp.float32),
        ],
    )(deg, mm1)


def _mm_body(x_ref, w_ref, out_ref):
    out_ref[...] = jnp.dot(x_ref[...], w_ref[...],
                           preferred_element_type=jnp.float32)


def _mm(x, w):
    m, k = x.shape
    n = w.shape[1]
    return pl.pallas_call(
        _mm_body,
        grid=(m // BM,),
        in_specs=[
            pl.BlockSpec((BM, k), lambda i: (i, 0)),
            pl.BlockSpec((k, n), lambda i: (0, 0)),
        ],
        out_specs=pl.BlockSpec((BM, n), lambda i: (i, 0)),
        out_shape=jax.ShapeDtypeStruct((m, n), jnp.float32),
    )(x, w)


def _epilogue_body(nw, post, dinv_ref, a0_ref, a1_ref, p_ref, b_ref, *rest):
    w_refs = rest[:nw]
    out_refs = rest[nw:]
    dinv = dinv_ref[...]
    h = dinv * (a0_ref[0] + a1_ref[0] + p_ref[...]) + b_ref[...]
    h = jnp.maximum(h, 0.0)
    if nw == 0:
        out_refs[0][...] = dinv * h if post else h
    else:
        for w_ref, o_ref in zip(w_refs, out_refs):
            t = jnp.dot(h, w_ref[...], preferred_element_type=jnp.float32)
            o_ref[...] = dinv * t


def _epilogue(dinv, a, p, b, ws, post=False):
    m, d = p.shape
    nw = len(ws)
    outs = ([jax.ShapeDtypeStruct((m, d), jnp.float32)] if nw == 0 else
            [jax.ShapeDtypeStruct((m, w.shape[1]), jnp.float32) for w in ws])
    w_specs = [pl.BlockSpec(w.shape, lambda i: (0, 0)) for w in ws]
    out_specs = ([pl.BlockSpec((BM, d), lambda i: (i, 0))] if nw == 0 else
                 [pl.BlockSpec((BM, w.shape[1]), lambda i: (i, 0)) for w in ws])
    return pl.pallas_call(
        functools.partial(_epilogue_body, nw, post),
        grid=(m // BM,),
        in_specs=[
            pl.BlockSpec((BM, 1), lambda i: (i, 0)),
            pl.BlockSpec((1, BM, d), lambda i: (0, i, 0)),
            pl.BlockSpec((1, BM, d), lambda i: (1, i, 0)),
            pl.BlockSpec((BM, d), lambda i: (i, 0)),
            pl.BlockSpec((1, d), lambda i: (0, 0)),
            *w_specs,
        ],
        out_specs=out_specs,
        out_shape=outs,
    )(dinv, a, a, p, b, *ws)


def _heads_body(heads, dinv_ref, a0_ref, a1_ref, q_ref, *rest):
    nh = len(heads)
    w_refs = rest[:nh]
    b_refs = rest[nh:2 * nh]
    out_refs = rest[2 * nh:]
    dinv = dinv_ref[...]
    B = a0_ref[0] + a1_ref[0] + q_ref[...]
    for (post, _), w_ref, b_ref, o_ref in zip(heads, w_refs, b_refs, out_refs):
        t = jnp.maximum(dinv * jnp.dot(B, w_ref[...],
                                       preferred_element_type=jnp.float32)
                        + b_ref[...], 0.0)
        o_ref[...] = dinv * t if post else t


def _heads(dinv, a, q, heads_ws):
    m, d = q.shape
    heads = [(post, w.shape[1]) for post, w, _ in heads_ws]
    ws = [w for _, w, _ in heads_ws]
    bs = [b for _, _, b in heads_ws]
    return pl.pallas_call(
        functools.partial(_heads_body, heads),
        grid=(m // BM,),
        in_specs=[
            pl.BlockSpec((BM, 1), lambda i: (i, 0)),
            pl.BlockSpec((1, BM, d), lambda i: (0, i, 0)),
            pl.BlockSpec((1, BM, d), lambda i: (1, i, 0)),
            pl.BlockSpec((BM, d), lambda i: (i, 0)),
            *[pl.BlockSpec(w.shape, lambda i: (0, 0)) for w in ws],
            *[pl.BlockSpec((1, b.shape[1]), lambda i: (0, 0)) for b in bs],
        ],
        out_specs=[pl.BlockSpec((BM, n), lambda i: (i, 0)) for _, n in heads],
        out_shape=[jax.ShapeDtypeStruct((m, n), jnp.float32) for _, n in heads],
    )(dinv, a, a, q, *ws, *bs)


BN = 2048


def _sst_body(sr_ref, sc_ref, out_ref):
    out_ref[...] = lax.dot_general(
        sr_ref[...], sc_ref[...],
        dimension_numbers=(((1,), (1,)), ((), ())),
        preferred_element_type=jnp.float32,
    )


def _s_st(s):
    m = s.shape[0]
    d = s.shape[1]
    return pl.pallas_call(
        _sst_body,
        grid=(m // BM, pl.cdiv(m, BN)),
        in_specs=[
            pl.BlockSpec((BM, d), lambda i, j: (i, 0)),
            pl.BlockSpec((BN, d), lambda i, j: (j, 0)),
        ],
        out_specs=pl.BlockSpec((BM, BN), lambda i, j: (i, j)),
        out_shape=jax.ShapeDtypeStruct((m, m), jnp.float32),
    )(s, s)


def kernel(x, edge_index, label, prior_labels, W1, b1, W2, b2, W3, b3, W4, b4, W5, b5):
    ei = edge_index.astype(jnp.int32)
    CP = 125
    e4 = ei.reshape(2, NW, EW // CP, CP).transpose(1, 2, 0, 3)
    dstdeg = ei[1].reshape(NW, NCH_DEG, C_DEG)

    z1 = jnp.zeros((N_PAD,), jnp.float32)
    z64 = jnp.zeros((N_PAD, D_HID), jnp.float32)

    ones_blk = jnp.ones((C_DEG,), jnp.float32)
    deg = _make_degree()(dstdeg, ones_blk, z1)
    mm1 = _mm(x, W1)
    dinv, p1 = _dinv_scale(deg, mm1)

    prop = _make_propagate4(D_HID, CP)

    b1r = b1.reshape(1, -1)
    b2r = b2.reshape(1, -1)
    b3r = b3.reshape(1, -1)
    b4r = b4.reshape(1, -1)
    b5r = b5.reshape(1, -1)

    a1 = prop(p1, e4, z64)
    (p2,) = _epilogue(dinv, a1, p1, b1r, [W2])
    a2 = prop(p2, e4, z64)
    (q2,) = _epilogue(dinv, a2, p2, b2r, [], post=True)

    aq2 = prop(q2, e4, z64)
    q4, s = _heads(dinv, aq2, q2, [(True, W3, b3r), (False, W5, b5r)])
    struct = _s_st(s)

    aq4 = prop(q4, e4, z64)
    (x_hat,) = _heads(dinv, aq4, q4, [(False, W4, b4r)])

    return (struct, x_hat, edge_index)
